# baseline (speedup 1.0000x reference)
import jax
import jax.numpy as jnp
from jax import lax
from jax.experimental import pallas as pl
from jax.experimental.pallas import tpu as pltpu
from jax.experimental.pallas import tpu_sc as plsc

B = 4096 * 200
D = 64
DP = 128
SCALE = 8.0

NW = 32
B_PER_W = B // NW
CHUNK = 200
N_CHUNKS = B_PER_W // CHUNK
LANES = 16
UNROLL = 4


def _emb_kernel(token_hbm, table_hbm, out_hbm,
                idx_all, buf0, buf1, obuf0, obuf1, g0, g1, s0, s1):
    wid = lax.axis_index("s") * 2 + lax.axis_index("c")
    base = wid * B_PER_W
    pltpu.sync_copy(token_hbm.at[pl.ds(base, B_PER_W)], idx_all)

    bufs = (buf0, buf1)
    obufs = (obuf0, obuf1)
    gsem = (g0, g1)
    ssem = (s0, s1)

    def gather_start(c, b):
        pltpu.async_copy(
            table_hbm.at[idx_all.at[pl.ds(c * CHUNK, CHUNK)]], bufs[b], gsem[b])

    def gather_wait(c, b):
        pltpu.make_async_copy(
            table_hbm.at[idx_all.at[pl.ds(c * CHUNK, CHUNK)]], bufs[b], gsem[b]).wait()

    def out_ref(c):
        off = pl.multiple_of(base + c * CHUNK, 8)
        return out_hbm.at[pl.ds(off, CHUNK)]

    def store_start(c, b):
        pltpu.async_copy(obufs[b], out_ref(c), ssem[b])

    def store_wait(c, b):
        pltpu.make_async_copy(obufs[b], out_ref(c), ssem[b]).wait()

    def scale_to_obuf(b):
        buf = bufs[b]
        obuf = obufs[b]

        def body(i, carry):
            for r in range(UNROLL):
                for j in range(D // LANES):
                    sl = pl.ds(j * LANES, LANES)
                    obuf[i * UNROLL + r, sl] = buf[i * UNROLL + r, sl] * SCALE
            return carry
        lax.fori_loop(0, CHUNK // UNROLL, body, 0)

    gather_start(0, 0)

    @pl.loop(0, N_CHUNKS, step=2)
    def _(c0):
        for b in range(2):
            c = c0 + b

            @pl.when(c + 1 < N_CHUNKS)
            def _():
                gather_start(c + 1, 1 - b)

            gather_wait(c, b)

            @pl.when(c >= 2)
            def _():
                store_wait(c - 2, b)

            scale_to_obuf(b)
            store_start(c, b)

    store_wait(N_CHUNKS - 2, 0)
    store_wait(N_CHUNKS - 1, 1)


@jax.jit
def kernel(token, table):
    tok = token.reshape(-1)
    tab_pad = jnp.pad(table.T, ((0, DP - D), (0, 0))).T
    mesh = plsc.VectorSubcoreMesh(core_axis_name="c", subcore_axis_name="s")
    out = pl.kernel(
        _emb_kernel,
        mesh=mesh,
        out_type=jax.ShapeDtypeStruct((B, D), jnp.float32),
        scratch_types=[
            pltpu.VMEM((B_PER_W,), jnp.int32),
            pltpu.VMEM((CHUNK, DP), jnp.float32),
            pltpu.VMEM((CHUNK, DP), jnp.float32),
            pltpu.VMEM((CHUNK, D), jnp.float32),
            pltpu.VMEM((CHUNK, D), jnp.float32),
            pltpu.SemaphoreType.DMA,
            pltpu.SemaphoreType.DMA,
            pltpu.SemaphoreType.DMA,
            pltpu.SemaphoreType.DMA,
        ],
    )(tok, tab_pad)
    return out.reshape(token.shape + (D,))


# speedup vs baseline: 1.0025x; 1.0025x over previous
import jax
import jax.numpy as jnp
from jax import lax
from jax.experimental import pallas as pl
from jax.experimental.pallas import tpu as pltpu
from jax.experimental.pallas import tpu_sc as plsc

B = 4096 * 200
D = 64
DP = 128
SCALE = 8.0

NW = 32
B_PER_W = B // NW
CHUNK = 200
N_CHUNKS = B_PER_W // CHUNK
LANES = 16
UNROLL = 8


def _emb_kernel(token_hbm, table_hbm, out_hbm,
                idx_all, buf0, buf1, obuf0, obuf1, g0, g1, s0, s1):
    wid = lax.axis_index("s") * 2 + lax.axis_index("c")
    base = wid * B_PER_W
    pltpu.sync_copy(token_hbm.at[pl.ds(base, B_PER_W)], idx_all)

    bufs = (buf0, buf1)
    obufs = (obuf0, obuf1)
    gsem = (g0, g1)
    ssem = (s0, s1)

    def gather_start(c, b):
        pltpu.async_copy(
            table_hbm.at[idx_all.at[pl.ds(c * CHUNK, CHUNK)]], bufs[b], gsem[b])

    def gather_wait(c, b):
        pltpu.make_async_copy(
            table_hbm.at[idx_all.at[pl.ds(c * CHUNK, CHUNK)]], bufs[b], gsem[b]).wait()

    def out_ref(c):
        off = pl.multiple_of(base + c * CHUNK, 8)
        return out_hbm.at[pl.ds(off, CHUNK)]

    def store_start(c, b):
        pltpu.async_copy(obufs[b], out_ref(c), ssem[b])

    def store_wait(c, b):
        pltpu.make_async_copy(obufs[b], out_ref(c), ssem[b]).wait()

    def scale_to_obuf(b):
        buf = bufs[b]
        obuf = obufs[b]

        def body(i, carry):
            for r in range(UNROLL):
                for j in range(D // LANES):
                    sl = pl.ds(j * LANES, LANES)
                    obuf[i * UNROLL + r, sl] = buf[i * UNROLL + r, sl] * SCALE
            return carry
        lax.fori_loop(0, CHUNK // UNROLL, body, 0)

    gather_start(0, 0)

    @pl.loop(0, N_CHUNKS, step=2)
    def _(c0):
        for b in range(2):
            c = c0 + b

            @pl.when(c + 1 < N_CHUNKS)
            def _():
                gather_start(c + 1, 1 - b)

            gather_wait(c, b)

            @pl.when(c >= 2)
            def _():
                store_wait(c - 2, b)

            scale_to_obuf(b)
            store_start(c, b)

    store_wait(N_CHUNKS - 2, 0)
    store_wait(N_CHUNKS - 1, 1)


@jax.jit
def kernel(token, table):
    tok = token.reshape(-1)
    tab_pad = jnp.pad(table, ((0, 0), (0, DP - D)))
    mesh = plsc.VectorSubcoreMesh(core_axis_name="c", subcore_axis_name="s")
    out = pl.kernel(
        _emb_kernel,
        mesh=mesh,
        out_type=jax.ShapeDtypeStruct((B, D), jnp.float32),
        scratch_types=[
            pltpu.VMEM((B_PER_W,), jnp.int32),
            pltpu.VMEM((CHUNK, DP), jnp.float32),
            pltpu.VMEM((CHUNK, DP), jnp.float32),
            pltpu.VMEM((CHUNK, D), jnp.float32),
            pltpu.VMEM((CHUNK, D), jnp.float32),
            pltpu.SemaphoreType.DMA,
            pltpu.SemaphoreType.DMA,
            pltpu.SemaphoreType.DMA,
            pltpu.SemaphoreType.DMA,
        ],
    )(tok, tab_pad)
    return out.reshape(token.shape + (D,))
